# baseline (speedup 1.0000x reference)
import functools

import jax
import jax.numpy as jnp
from jax import lax
from jax.experimental import pallas as pl
from jax.experimental.pallas import tpu as pltpu
from jax.experimental.pallas import tpu_sc as plsc

NC = 2
NS = 16
NW = NC * NS
B = 128


def _sc_segment_sum(x, edges, n_pad, full_ct, last_ct):
    n_nodes, d = x.shape
    rows_pt = n_pad // NS
    rchunk = 40
    nrc = rows_pt // rchunk
    half = full_ct // 2
    n_edges = edges.shape[1]
    assert full_ct % 2 == 0 and last_ct % 2 == 0 and 0 < last_ct <= half
    assert rows_pt % rchunk == 0 and d % 16 == 0 and rchunk % 8 == 0

    mesh = plsc.VectorSubcoreMesh(core_axis_name="c", subcore_axis_name="s")

    @functools.partial(
        pl.kernel,
        mesh=mesh,
        out_type=jax.ShapeDtypeStruct((NC, n_pad, d), jnp.float32),
        scratch_types=[
            pltpu.VMEM((2, half * B), jnp.int32),
            pltpu.VMEM((B, d), jnp.float32),
            pltpu.VMEM((B, d), jnp.float32),
            pltpu.VMEM_SHARED((n_pad, d), jnp.float32),
            pltpu.SemaphoreType.DMA,
            pltpu.SemaphoreType.DMA,
            pltpu.SemaphoreType.DMA,
            pltpu.SemaphoreType.DMA,
            pltpu.SemaphoreType.DMA,
        ],
    )
    def k(x_hbm, edges_hbm, out_hbm,
          ev, rows_a, rows_b, acc, sem_ga, sem_gb, sem_sa, sem_sb, sem_ix):
        c = lax.axis_index("c")
        s = lax.axis_index("s")
        wid = s * NC + c
        ebase = jnp.minimum(wid * (full_ct * B), n_edges - half * B)
        bounce = rows_a.at[pl.ds(0, rchunk)]

        def stage_indices(j0):
            return pltpu.async_copy(
                edges_hbm.at[pl.ds(0, 2), pl.ds(ebase + j0 * B, half * B)],
                ev, sem_ix)

        cp0 = stage_indices(0)

        zeros = jnp.zeros((16,), jnp.float32)

        def zrow(i, carry):
            def zcol(j, carry2):
                rows_a[i, pl.ds(j * 16, 16)] = zeros
                return carry2
            return lax.fori_loop(0, d // 16, zcol, carry)

        lax.fori_loop(0, rchunk, zrow, 0)
        row0 = s * rows_pt
        for r in range(nrc):
            pltpu.sync_copy(bounce, acc.at[pl.ds(row0 + r * rchunk, rchunk)])
        cp0.wait()
        plsc.subcore_barrier()

        def src_ix(j):
            return ev.at[0, pl.ds(j * B, B)]

        def dst_ix(j):
            return ev.at[1, pl.ds(j * B, B)]

        def gather(j, buf, sem):
            pltpu.async_copy(x_hbm.at[src_ix(j)], buf, sem)

        def wait_gather(j, buf, sem):
            pltpu.make_async_copy(x_hbm.at[src_ix(j)], buf, sem).wait()

        def scatter(j, buf, sem):
            pltpu.async_copy(buf, acc.at[dst_ix(j)], sem, add=True)

        def wait_scatter(j, buf, sem):
            pltpu.make_async_copy(buf, acc.at[dst_ix(j)], sem).wait()

        def run_stage(nst, joff=0):
            gather(joff + 0, rows_a, sem_ga)
            gather(joff + 1, rows_b, sem_gb)

            def body(m, carry):
                j = joff + 2 * m
                wait_gather(j, rows_a, sem_ga)
                scatter(j, rows_a, sem_sa)
                wait_gather(j + 1, rows_b, sem_gb)
                scatter(j + 1, rows_b, sem_sb)
                wait_scatter(j, rows_a, sem_sa)
                gather(j + 2, rows_a, sem_ga)
                wait_scatter(j + 1, rows_b, sem_sb)
                gather(j + 3, rows_b, sem_gb)
                return carry

            lax.fori_loop(0, nst // 2 - 1, body, 0)
            j = joff + nst - 2
            wait_gather(j, rows_a, sem_ga)
            scatter(j, rows_a, sem_sa)
            wait_gather(j + 1, rows_b, sem_gb)
            scatter(j + 1, rows_b, sem_sb)
            wait_scatter(j, rows_a, sem_sa)
            wait_scatter(j + 1, rows_b, sem_sb)

        @pl.when(wid != NW - 1)
        def _full():
            run_stage(half)
            stage_indices(half).wait()
            run_stage(half)

        @pl.when(wid == NW - 1)
        def _short():
            run_stage(last_ct, joff=half - last_ct)

        plsc.subcore_barrier()

        pltpu.sync_copy(acc.at[pl.ds(row0, rows_pt)],
                        out_hbm.at[c, pl.ds(row0, rows_pt)])

    return k(x, edges)


def _tc_transform(partials, w, n_nodes):
    d = partials.shape[2]
    r = 1000
    grid = (n_nodes // r,)

    def body(p_ref, w_ref, o_ref):
        o_ref[...] = jnp.dot(p_ref[0] + p_ref[1], w_ref[...],
                             preferred_element_type=jnp.float32)

    return pl.pallas_call(
        body,
        grid=grid,
        in_specs=[
            pl.BlockSpec((2, r, d), lambda i: (0, i, 0)),
            pl.BlockSpec((d, d), lambda i: (0, 0)),
        ],
        out_specs=pl.BlockSpec((r, d), lambda i: (i, 0)),
        out_shape=jax.ShapeDtypeStruct((n_nodes, d), jnp.float32),
    )(partials, w)


def kernel(x, edge_index, W):
    n_nodes = x.shape[0]
    n_edges = edge_index.shape[1]
    n_pad = ((n_nodes + NS * 128 - 1) // (NS * 128)) * (NS * 128)
    nchunks = n_edges // B
    assert n_edges % B == 0
    full_ct = -(-nchunks // NW)
    if full_ct % 2:
        full_ct += 1
    last_ct = nchunks - (NW - 1) * full_ct
    assert 0 < last_ct <= full_ct and last_ct % 2 == 0
    edges = edge_index.astype(jnp.int32)
    partials = _sc_segment_sum(x, edges, n_pad, full_ct, last_ct)
    return _tc_transform(partials, W, n_nodes)


# speedup vs baseline: 1.2690x; 1.2690x over previous
import functools

import jax
import jax.numpy as jnp
from jax import lax
from jax.experimental import pallas as pl
from jax.experimental.pallas import tpu as pltpu
from jax.experimental.pallas import tpu_sc as plsc

NC = 2
NS = 16
NW = NC * NS
B = 128


def _sc_segment_sum(x, edges, n_pad, full_ct, last_ct):
    n_nodes, d = x.shape
    rows_pt = n_pad // NS
    rchunk = 40
    nrc = rows_pt // rchunk
    half = full_ct // 2
    n_edges = edges.shape[1]
    assert full_ct % 2 == 0 and last_ct % 2 == 0 and 0 < last_ct <= half
    assert rows_pt % rchunk == 0 and d % 16 == 0 and rchunk % 8 == 0

    mesh = plsc.VectorSubcoreMesh(core_axis_name="c", subcore_axis_name="s")

    @functools.partial(
        pl.kernel,
        mesh=mesh,
        out_type=jax.ShapeDtypeStruct((NC, n_pad, d), jnp.float32),
        scratch_types=[
            pltpu.VMEM((2, half * B), jnp.int32),
            pltpu.VMEM((B, d), jnp.float32),
            pltpu.VMEM((B, d), jnp.float32),
            pltpu.VMEM_SHARED((n_pad, d), jnp.float32),
            pltpu.SemaphoreType.DMA,
            pltpu.SemaphoreType.DMA,
            pltpu.SemaphoreType.DMA,
            pltpu.SemaphoreType.DMA,
            pltpu.SemaphoreType.DMA,
        ],
    )
    def k(x_hbm, edges_hbm, out_hbm,
          ev, rows_a, rows_b, acc, sem_ga, sem_gb, sem_sa, sem_sb, sem_ix):
        c = lax.axis_index("c")
        s = lax.axis_index("s")
        wid = s * NC + c
        ebase = jnp.minimum(wid * (full_ct * B), n_edges - half * B)
        bounce = rows_a.at[pl.ds(0, rchunk)]

        def stage_indices(j0):
            return pltpu.async_copy(
                edges_hbm.at[pl.ds(0, 2), pl.ds(ebase + j0 * B, half * B)],
                ev, sem_ix)

        cp0 = stage_indices(0)

        zeros = jnp.zeros((16,), jnp.float32)

        def zrow(i, carry):
            def zcol(j, carry2):
                rows_a[i, pl.ds(j * 16, 16)] = zeros
                return carry2
            return lax.fori_loop(0, d // 16, zcol, carry)

        lax.fori_loop(0, rchunk, zrow, 0)
        row0 = s * rows_pt
        for r in range(nrc):
            pltpu.sync_copy(bounce, acc.at[pl.ds(row0 + r * rchunk, rchunk)])
        cp0.wait()
        plsc.subcore_barrier()

        def src_ix(j):
            return ev.at[0, pl.ds(j * B, B)]

        def dst_ix(j):
            return ev.at[1, pl.ds(j * B, B)]

        def gather(j, buf, sem):
            pltpu.async_copy(x_hbm.at[src_ix(j)], buf, sem)

        def wait_gather(j, buf, sem):
            pltpu.make_async_copy(x_hbm.at[src_ix(j)], buf, sem).wait()

        def scatter(j, buf):
            pltpu.sync_copy(buf, acc.at[dst_ix(j)], add=True)

        def run_stage(nst, joff=0):
            gather(joff + 0, rows_a, sem_ga)
            gather(joff + 1, rows_b, sem_gb)

            def body(m, carry):
                j = joff + 2 * m
                wait_gather(j, rows_a, sem_ga)
                scatter(j, rows_a)
                gather(j + 2, rows_a, sem_ga)
                wait_gather(j + 1, rows_b, sem_gb)
                scatter(j + 1, rows_b)
                gather(j + 3, rows_b, sem_gb)
                return carry

            lax.fori_loop(0, nst // 2 - 1, body, 0)
            j = joff + nst - 2
            wait_gather(j, rows_a, sem_ga)
            scatter(j, rows_a)
            wait_gather(j + 1, rows_b, sem_gb)
            scatter(j + 1, rows_b)

        @pl.when(wid != NW - 1)
        def _full():
            run_stage(half)
            stage_indices(half).wait()
            run_stage(half)

        @pl.when(wid == NW - 1)
        def _short():
            run_stage(last_ct, joff=half - last_ct)

        plsc.subcore_barrier()

        pltpu.sync_copy(acc.at[pl.ds(row0, rows_pt)],
                        out_hbm.at[c, pl.ds(row0, rows_pt)])

    return k(x, edges)


def _tc_transform(partials, w, n_nodes):
    d = partials.shape[2]
    r = 1000
    grid = (n_nodes // r,)

    def body(p_ref, w_ref, o_ref):
        o_ref[...] = jnp.dot(p_ref[0] + p_ref[1], w_ref[...],
                             preferred_element_type=jnp.float32)

    return pl.pallas_call(
        body,
        grid=grid,
        in_specs=[
            pl.BlockSpec((2, r, d), lambda i: (0, i, 0)),
            pl.BlockSpec((d, d), lambda i: (0, 0)),
        ],
        out_specs=pl.BlockSpec((r, d), lambda i: (i, 0)),
        out_shape=jax.ShapeDtypeStruct((n_nodes, d), jnp.float32),
    )(partials, w)


def kernel(x, edge_index, W):
    n_nodes = x.shape[0]
    n_edges = edge_index.shape[1]
    n_pad = ((n_nodes + NS * 128 - 1) // (NS * 128)) * (NS * 128)
    nchunks = n_edges // B
    assert n_edges % B == 0
    full_ct = -(-nchunks // NW)
    if full_ct % 2:
        full_ct += 1
    last_ct = nchunks - (NW - 1) * full_ct
    assert 0 < last_ct <= full_ct and last_ct % 2 == 0
    edges = edge_index.astype(jnp.int32)
    partials = _sc_segment_sum(x, edges, n_pad, full_ct, last_ct)
    return _tc_transform(partials, W, n_nodes)


# speedup vs baseline: 1.3020x; 1.0260x over previous
import functools

import jax
import jax.numpy as jnp
from jax import lax
from jax.experimental import pallas as pl
from jax.experimental.pallas import tpu as pltpu
from jax.experimental.pallas import tpu_sc as plsc

NC = 2
NS = 16
NW = NC * NS
B = 128


def _sc_segment_sum(x, edges, n_pad, full_ct, last_ct):
    n_nodes, d = x.shape
    rows_pt = n_pad // NS
    rchunk = 40
    nrc = rows_pt // rchunk
    half = full_ct // 2
    n_edges = edges.shape[1]
    assert full_ct % 2 == 0 and last_ct % 2 == 0 and 0 < last_ct <= half
    assert rows_pt % rchunk == 0 and d % 16 == 0 and rchunk % 8 == 0

    mesh = plsc.VectorSubcoreMesh(core_axis_name="c", subcore_axis_name="s")

    @functools.partial(
        pl.kernel,
        mesh=mesh,
        out_type=jax.ShapeDtypeStruct((NC, n_pad, d), jnp.float32),
        scratch_types=[
            pltpu.VMEM((2, half * B), jnp.int32),
            pltpu.VMEM((B, d), jnp.float32),
            pltpu.VMEM((B, d), jnp.float32),
            pltpu.VMEM_SHARED((n_pad, d), jnp.float32),
            pltpu.SemaphoreType.DMA,
            pltpu.SemaphoreType.DMA,
            pltpu.SemaphoreType.DMA,
            pltpu.SemaphoreType.DMA,
            pltpu.SemaphoreType.DMA,
        ],
    )
    def k(x_hbm, edges_hbm, out_hbm,
          ev, rows_a, rows_b, acc, sem_ga, sem_gb, sem_sa, sem_sb, sem_ix):
        c = lax.axis_index("c")
        s = lax.axis_index("s")
        wid = s * NC + c
        ebase = jnp.minimum(wid * (full_ct * B), n_edges - half * B)
        bounce = rows_a.at[pl.ds(0, rchunk)]

        def stage_indices(j0):
            return pltpu.async_copy(
                edges_hbm.at[pl.ds(0, 2), pl.ds(ebase + j0 * B, half * B)],
                ev, sem_ix)

        cp0 = stage_indices(0)

        zeros = jnp.zeros((16,), jnp.float32)

        def zrow(i, carry):
            def zcol(j, carry2):
                rows_a[i, pl.ds(j * 16, 16)] = zeros
                return carry2
            return lax.fori_loop(0, d // 16, zcol, carry)

        lax.fori_loop(0, rchunk, zrow, 0)
        row0 = s * rows_pt
        for r in range(nrc):
            pltpu.sync_copy(bounce, acc.at[pl.ds(row0 + r * rchunk, rchunk)])
        cp0.wait()
        plsc.subcore_barrier()

        def src_ix(j):
            return ev.at[0, pl.ds(j * B, B)]

        def dst_ix(j):
            return ev.at[1, pl.ds(j * B, B)]

        def gather(j, buf, sem):
            pltpu.async_copy(x_hbm.at[src_ix(j)], buf, sem)

        def wait_gather(j, buf, sem):
            pltpu.make_async_copy(x_hbm.at[src_ix(j)], buf, sem).wait()

        def scatter(j, buf):
            pltpu.sync_copy(buf, acc.at[dst_ix(j)], add=True)

        def run_stage(nst, joff=0):
            gather(joff + 0, rows_a, sem_ga)
            gather(joff + 1, rows_b, sem_gb)

            def body(m, carry):
                j = joff + 2 * m
                wait_gather(j, rows_a, sem_ga)
                scatter(j, rows_a)
                gather(j + 2, rows_a, sem_ga)
                wait_gather(j + 1, rows_b, sem_gb)
                scatter(j + 1, rows_b)
                gather(j + 3, rows_b, sem_gb)
                return carry

            lax.fori_loop(0, nst // 2 - 1, body, 0)
            j = joff + nst - 2
            wait_gather(j, rows_a, sem_ga)
            scatter(j, rows_a)
            wait_gather(j + 1, rows_b, sem_gb)
            scatter(j + 1, rows_b)

        @pl.when(wid != NW - 1)
        def _full():
            run_stage(half)
            stage_indices(half).wait()
            run_stage(half)

        @pl.when(wid == NW - 1)
        def _short():
            run_stage(last_ct, joff=half - last_ct)

        plsc.subcore_barrier()

        pltpu.sync_copy(acc.at[pl.ds(row0, rows_pt)],
                        out_hbm.at[c, pl.ds(row0, rows_pt)])

    return k(x, edges)


def _tc_transform(partials, w, n_nodes):
    d = partials.shape[2]
    r = 2000
    grid = (n_nodes // r,)

    def body(p_ref, w_ref, o_ref):
        o_ref[...] = jnp.dot(p_ref[0] + p_ref[1], w_ref[...],
                             preferred_element_type=jnp.float32)

    return pl.pallas_call(
        body,
        grid=grid,
        in_specs=[
            pl.BlockSpec((2, r, d), lambda i: (0, i, 0)),
            pl.BlockSpec((d, d), lambda i: (0, 0)),
        ],
        out_specs=pl.BlockSpec((r, d), lambda i: (i, 0)),
        out_shape=jax.ShapeDtypeStruct((n_nodes, d), jnp.float32),
    )(partials, w)


def kernel(x, edge_index, W):
    n_nodes = x.shape[0]
    n_edges = edge_index.shape[1]
    n_pad = ((n_nodes + NS * 128 - 1) // (NS * 128)) * (NS * 128)
    nchunks = n_edges // B
    assert n_edges % B == 0
    full_ct = -(-nchunks // NW)
    if full_ct % 2:
        full_ct += 1
    last_ct = nchunks - (NW - 1) * full_ct
    assert 0 < last_ct <= full_ct and last_ct % 2 == 0
    edges = edge_index.astype(jnp.int32)
    partials = _sc_segment_sum(x, edges, n_pad, full_ct, last_ct)
    return _tc_transform(partials, W, n_nodes)
